# baseline (speedup 1.0000x reference)
import functools

import jax
import jax.numpy as jnp
from jax import lax
from jax.experimental import pallas as pl
from jax.experimental.pallas import tpu as pltpu
from jax.experimental.pallas import tpu_sc as plsc

N = 10000
E = 320000
FE = 16
D = 128
G = 64
C = 12
L = 6

NC = 2
NS = 16
NW = NC * NS
EW = E // NW
K = 40
EWP = 10080
CH = EWP // K
PASS = 4
CPP = CH // PASS
E2 = NW * EWP
N2 = N + 8
SRPT = 624
TAILZ = N2 - SRPT * NS
TAIL = N - SRPT * NS

_CT = (((1,), (1,)), ((), ()))


def _edge_mlp_body(ea_ref, w1_ref, b1_ref, w2_ref, b2_ref, out_ref):
    h1 = lax.dot_general(ea_ref[...], w1_ref[...], _CT,
                         preferred_element_type=jnp.float32)
    h1 = jnp.maximum(h1 + b1_ref[...], 0.0)
    h2 = lax.dot_general(h1, w2_ref[...], _CT,
                         preferred_element_type=jnp.float32)
    out_ref[...] = h2 + b2_ref[...]


def _edge_mlp(edge_attr, w1, b1, w2, b2):
    BE = 3840
    return pl.pallas_call(
        _edge_mlp_body,
        grid=(E2 // BE,),
        in_specs=[
            pl.BlockSpec((BE, FE), lambda i: (i, 0)),
            pl.BlockSpec((D, FE), lambda i: (0, 0)),
            pl.BlockSpec((1, D), lambda i: (0, 0)),
            pl.BlockSpec((D, D), lambda i: (0, 0)),
            pl.BlockSpec((1, D), lambda i: (0, 0)),
        ],
        out_specs=pl.BlockSpec((BE, D), lambda i: (i, 0)),
        out_shape=jax.ShapeDtypeStruct((E2, D), jnp.float32),
    )(edge_attr, w1, b1.reshape(1, D), w2, b2.reshape(1, D))


@functools.cache
def _sc_gather_scatter_fn():
    mesh = plsc.VectorSubcoreMesh(
        core_axis_name="c", subcore_axis_name="s",
        num_cores=NC, num_subcores=NS)

    @functools.partial(
        pl.kernel,
        out_type=jax.ShapeDtypeStruct((NC, N, D), jnp.float32),
        mesh=mesh,
        scratch_types=[
            pltpu.VMEM((CPP, K), jnp.int32),
            pltpu.VMEM((CPP, K), jnp.int32),
            pltpu.VMEM((3, K, D), jnp.float32),
            pltpu.VMEM((3, K, D), jnp.float32),
            pltpu.VMEM_SHARED((N2, D), jnp.float32),
            [pltpu.SemaphoreType.DMA] * 3,
            [pltpu.SemaphoreType.DMA] * 3,
            [pltpu.SemaphoreType.DMA] * 3,
        ],
    )
    def _sc_gather_scatter(h_hbm, emb_hbm, src_hbm, dst_hbm, zer_hbm, out_hbm,
                           srcv, dstv, rowsb, embb, aggr, gsem, esem, ssem):
        cid = lax.axis_index("c")
        sid = lax.axis_index("s")
        wid = cid * NS + sid
        start = pl.multiple_of(sid * SRPT, 8)
        pltpu.sync_copy(zer_hbm, aggr.at[pl.ds(start, SRPT)])

        @pl.when(sid == 0)
        def _zero_tail():
            pltpu.sync_copy(zer_hbm.at[pl.ds(0, TAILZ)],
                            aggr.at[pl.ds(SRPT * NS, TAILZ)])
        plsc.subcore_barrier()

        def _relu_add(rows, emb):
            def edge_body(e, carry2):
                for d8 in range(D // 16):
                    sl = pl.ds(d8 * 16, 16)
                    rows[e, sl] = jnp.maximum(rows[e, sl] + emb[e, sl], 0.0)
                return carry2

            lax.fori_loop(0, K, edge_body, 0)

        def pass_body(p, carry0):
            pltpu.sync_copy(src_hbm.at[wid, p], srcv)
            pltpu.sync_copy(dst_hbm.at[wid, p], dstv)
            base = (wid * CH + p * CPP) * K

            def issue_ge(c, b):
                pltpu.async_copy(h_hbm.at[srcv.at[c]], rowsb.at[b], gsem[b])
                pltpu.async_copy(emb_hbm.at[pl.ds(base + c * K, K)],
                                 embb.at[b], esem[b])

            def drain(sem, b):
                pltpu.make_async_copy(h_hbm.at[pl.ds(0, K)], rowsb.at[b],
                                      sem[b]).wait()

            issue_ge(0, 0)

            def triple_body(i, carry):
                for j in range(3):
                    c = 3 * i + j
                    nxt = (j + 1) % 3

                    @pl.when((c >= 2) & (c + 1 < CPP))
                    def _wait_prev_scatter():
                        drain(ssem, nxt)

                    @pl.when(c + 1 < CPP)
                    def _prefetch():
                        issue_ge(c + 1, nxt)

                    drain(gsem, j)
                    drain(esem, j)
                    _relu_add(rowsb.at[j], embb.at[j])
                    pltpu.async_copy(rowsb.at[j], aggr.at[dstv.at[c]],
                                     ssem[j], add=True)
                return carry

            lax.fori_loop(0, CPP // 3, triple_body, 0)
            for b in range(3):
                drain(ssem, b)
            return carry0

        lax.fori_loop(0, PASS, pass_body, 0)
        plsc.subcore_barrier()
        pltpu.sync_copy(aggr.at[pl.ds(start, SRPT)],
                        out_hbm.at[cid, pl.ds(start, SRPT)])

        @pl.when(sid == 0)
        def _flush_tail():
            pltpu.sync_copy(aggr.at[pl.ds(SRPT * NS, TAIL)],
                            out_hbm.at[cid, pl.ds(SRPT * NS, TAIL)])

    return _sc_gather_scatter


def _node_mlp_body(eps_ref, h_ref, a0_ref, a1_ref, w1_ref, b1_ref, w2_ref,
                   b2_ref, out_ref):
    z = (1.0 + eps_ref[0]) * h_ref[...] + a0_ref[...] + a1_ref[...]
    z1 = lax.dot_general(z, w1_ref[...], _CT,
                         preferred_element_type=jnp.float32)
    z1 = jnp.maximum(z1 + b1_ref[...], 0.0)
    z2 = lax.dot_general(z1, w2_ref[...], _CT,
                         preferred_element_type=jnp.float32)
    out_ref[...] = jnp.maximum(z2 + b2_ref[...], 0.0)


def _node_mlp(h, a0, a1, eps_l, w1, b1, w2, b2):
    BN = 2000
    return pl.pallas_call(
        _node_mlp_body,
        grid=(N // BN,),
        in_specs=[
            pl.BlockSpec(memory_space=pltpu.SMEM),
            pl.BlockSpec((BN, D), lambda i: (i, 0)),
            pl.BlockSpec((BN, D), lambda i: (i, 0)),
            pl.BlockSpec((BN, D), lambda i: (i, 0)),
            pl.BlockSpec((D, D), lambda i: (0, 0)),
            pl.BlockSpec((1, D), lambda i: (0, 0)),
            pl.BlockSpec((D, D), lambda i: (0, 0)),
            pl.BlockSpec((1, D), lambda i: (0, 0)),
        ],
        out_specs=pl.BlockSpec((BN, D), lambda i: (i, 0)),
        out_shape=jax.ShapeDtypeStruct((N, D), jnp.float32),
    )(eps_l.reshape(1), h, a0, a1, w1, b1.reshape(1, D), w2, b2.reshape(1, D))


def _s2s_body(h_ref, b_ref, wih_ref, whh_ref, bi_ref, bh_ref, f1w_ref,
              f1b_ref, f4w_ref, f4b_ref, out_ref):
    h = h_ref[...]
    gid = lax.broadcasted_iota(jnp.int32, (G, N), 0)
    maskb = gid == b_ref[...]
    q_star = jnp.zeros((G, 2 * D), jnp.float32)
    hs = jnp.zeros((G, D), jnp.float32)
    cs = jnp.zeros((G, D), jnp.float32)
    for _ in range(6):
        gates = (lax.dot_general(q_star, wih_ref[...], _CT,
                                 preferred_element_type=jnp.float32)
                 + lax.dot_general(hs, whh_ref[...], _CT,
                                   preferred_element_type=jnp.float32)
                 + bi_ref[...] + bh_ref[...])
        ig = jax.nn.sigmoid(gates[:, :D])
        fg = jax.nn.sigmoid(gates[:, D:2 * D])
        gg = jnp.tanh(gates[:, 2 * D:3 * D])
        og = jax.nn.sigmoid(gates[:, 3 * D:])
        cs = fg * cs + ig * gg
        hs = og * jnp.tanh(cs)
        s = lax.dot_general(hs, h, _CT,
                            preferred_element_type=jnp.float32)
        emax = jnp.max(jnp.where(maskb, s, -jnp.inf), axis=1, keepdims=True)
        a = jnp.exp(jnp.where(maskb, s - emax, -jnp.inf))
        denom = jnp.sum(a, axis=1, keepdims=True)
        a = a / (denom + 1e-16)
        r = lax.dot_general(a, h, (((1,), (0,)), ((), ())),
                            preferred_element_type=jnp.float32)
        q_star = jnp.concatenate([hs, r], axis=1)
    z = lax.dot_general(q_star, f1w_ref[...], _CT,
                        preferred_element_type=jnp.float32)
    z = jnp.maximum(z + f1b_ref[...], 0.0)
    out = lax.dot_general(z, f4w_ref[...], _CT,
                          preferred_element_type=jnp.float32)
    out_ref[...] = out + f4b_ref[...]


def _set2set(h, batch_row, lstm_Wih, lstm_Whh, lstm_bih, lstm_bhh,
             fc1_W, fc1_b, fc4_W, fc4_b):
    return pl.pallas_call(
        _s2s_body,
        out_shape=jax.ShapeDtypeStruct((G, C), jnp.float32),
    )(h, batch_row, lstm_Wih, lstm_Whh, lstm_bih.reshape(1, 4 * D),
      lstm_bhh.reshape(1, 4 * D), fc1_W, fc1_b.reshape(1, D), fc4_W,
      fc4_b.reshape(1, C))


def kernel(x, edge_index, edge_attr, batch, bW1, bb1, bW2, bb2, mW1, mb1,
           mW2, mb2, eps, lstm_Wih, lstm_Whh, lstm_bih, lstm_bhh, fc1_W,
           fc1_b, fc4_W, fc4_b):
    srcp = jnp.pad(edge_index[0].astype(jnp.int32).reshape(NW, EW),
                   ((0, 0), (0, EWP - EW)))
    dstp = jnp.pad(edge_index[1].astype(jnp.int32).reshape(NW, EW),
                   ((0, 0), (0, EWP - EW)), constant_values=N)
    src = srcp.reshape(NW, PASS, CPP, K)
    dst = dstp.reshape(NW, PASS, CPP, K)
    eap = jnp.pad(edge_attr.reshape(NW, EW, FE),
                  ((0, 0), (0, EWP - EW), (0, 0))).reshape(E2, FE)
    zer = jnp.zeros((SRPT, D), jnp.float32)
    batch_row = batch.astype(jnp.int32).reshape(1, N)
    h = x
    eembs = [_edge_mlp(eap, bW1[l], bb1[l], bW2[l], bb2[l])
             for l in range(L)]
    for l in range(L):
        parts = _sc_gather_scatter_fn()(h, eembs[l], src, dst, zer)
        h = _node_mlp(h, parts[0], parts[1], eps[l], mW1[l], mb1[l],
                      mW2[l], mb2[l])
    return _set2set(h, batch_row, lstm_Wih, lstm_Whh, lstm_bih, lstm_bhh,
                    fc1_W, fc1_b, fc4_W, fc4_b)


# speedup vs baseline: 1.1386x; 1.1386x over previous
import functools

import jax
import jax.numpy as jnp
from jax import lax
from jax.experimental import pallas as pl
from jax.experimental.pallas import tpu as pltpu
from jax.experimental.pallas import tpu_sc as plsc

N = 10000
E = 320000
FE = 16
D = 128
G = 64
C = 12
L = 6

NC = 2
NS = 16
NW = NC * NS
EW = E // NW
K = 40
CH = EW // K
PASS = 5
CPP = CH // PASS
SRPT = 624
TAIL = N - SRPT * NS

_CT = (((1,), (1,)), ((), ()))


def _edge_mlp_body(ea_ref, w1_ref, b1_ref, w2_ref, b2_ref, out_ref):
    h1 = lax.dot_general(ea_ref[...], w1_ref[...], _CT,
                         preferred_element_type=jnp.float32)
    h1 = jnp.maximum(h1 + b1_ref[...], 0.0)
    h2 = lax.dot_general(h1, w2_ref[...], _CT,
                         preferred_element_type=jnp.float32)
    out_ref[...] = h2 + b2_ref[...]


def _edge_mlp(edge_attr, w1, b1, w2, b2):
    BE = 4000
    return pl.pallas_call(
        _edge_mlp_body,
        grid=(E // BE,),
        in_specs=[
            pl.BlockSpec((BE, FE), lambda i: (i, 0)),
            pl.BlockSpec((D, FE), lambda i: (0, 0)),
            pl.BlockSpec((1, D), lambda i: (0, 0)),
            pl.BlockSpec((D, D), lambda i: (0, 0)),
            pl.BlockSpec((1, D), lambda i: (0, 0)),
        ],
        out_specs=pl.BlockSpec((BE, D), lambda i: (i, 0)),
        out_shape=jax.ShapeDtypeStruct((E, D), jnp.float32),
    )(edge_attr, w1, b1.reshape(1, D), w2, b2.reshape(1, D))


@functools.cache
def _sc_gather_scatter_fn():
    mesh = plsc.VectorSubcoreMesh(
        core_axis_name="c", subcore_axis_name="s",
        num_cores=NC, num_subcores=NS)

    @functools.partial(
        pl.kernel,
        out_type=jax.ShapeDtypeStruct((NC, N, D), jnp.float32),
        mesh=mesh,
        scratch_types=[
            pltpu.VMEM((CPP, K), jnp.int32),
            pltpu.VMEM((CPP, K), jnp.int32),
            pltpu.VMEM((K, D), jnp.float32),
            pltpu.VMEM((K, D), jnp.float32),
            pltpu.VMEM((K, D), jnp.float32),
            pltpu.VMEM((K, D), jnp.float32),
            pltpu.VMEM_SHARED((N, D), jnp.float32),
            pltpu.SemaphoreType.DMA,
            pltpu.SemaphoreType.DMA,
            pltpu.SemaphoreType.DMA,
            pltpu.SemaphoreType.DMA,
            pltpu.SemaphoreType.DMA,
            pltpu.SemaphoreType.DMA,
        ],
    )
    def _sc_gather_scatter(h_hbm, emb_hbm, src_hbm, dst_hbm, zer_hbm, out_hbm,
                           srcv, dstv, rows0, rows1, emb0, emb1, aggr,
                           gsem0, gsem1, esem0, esem1, ssem0, ssem1):
        cid = lax.axis_index("c")
        sid = lax.axis_index("s")
        wid = cid * NS + sid
        start = pl.multiple_of(sid * SRPT, 8)
        pltpu.sync_copy(zer_hbm, aggr.at[pl.ds(start, SRPT)])

        @pl.when(sid == 0)
        def _zero_tail():
            pltpu.sync_copy(zer_hbm.at[pl.ds(0, TAIL)],
                            aggr.at[pl.ds(SRPT * NS, TAIL)])
        plsc.subcore_barrier()

        def _relu_add(rows, emb):
            def edge_body(e, carry2):
                for d8 in range(D // 16):
                    sl = pl.ds(d8 * 16, 16)
                    rows[e, sl] = jnp.maximum(rows[e, sl] + emb[e, sl], 0.0)
                return carry2

            lax.fori_loop(0, K, edge_body, 0)

        def pass_body(p, carry0):
            pltpu.sync_copy(src_hbm.at[wid, p], srcv)
            pltpu.sync_copy(dst_hbm.at[wid, p], dstv)
            base = (wid * CH + p * CPP) * K

            def chunk0(c):
                return (pltpu.async_copy(h_hbm.at[srcv.at[c]], rows0, gsem0),
                        pltpu.async_copy(emb_hbm.at[pl.ds(base + c * K, K)],
                                         emb0, esem0))

            def drain(sem):
                pltpu.make_async_copy(h_hbm.at[pl.ds(0, K)], rows0,
                                      sem).wait()

            def group_body(i, carry):
                c0 = 2 * i
                c1 = c0 + 1

                @pl.when(i > 0)
                def _drain_prev_s0():
                    drain(ssem0)

                g0, e0 = chunk0(c0)

                @pl.when(i > 0)
                def _drain_prev_s1():
                    drain(ssem1)

                g1 = pltpu.async_copy(h_hbm.at[srcv.at[c1]], rows1, gsem1)
                e1 = pltpu.async_copy(emb_hbm.at[pl.ds(base + c1 * K, K)],
                                      emb1, esem1)
                g0.wait()
                e0.wait()
                _relu_add(rows0, emb0)
                pltpu.async_copy(rows0, aggr.at[dstv.at[c0]], ssem0,
                                 add=True)
                g1.wait()
                e1.wait()
                _relu_add(rows1, emb1)
                pltpu.async_copy(rows1, aggr.at[dstv.at[c1]], ssem1,
                                 add=True)
                return carry

            lax.fori_loop(0, CPP // 2, group_body, 0)
            drain(ssem0)
            drain(ssem1)
            return carry0

        lax.fori_loop(0, PASS, pass_body, 0)
        plsc.subcore_barrier()
        pltpu.sync_copy(aggr.at[pl.ds(start, SRPT)],
                        out_hbm.at[cid, pl.ds(start, SRPT)])

        @pl.when(sid == 0)
        def _flush_tail():
            pltpu.sync_copy(aggr.at[pl.ds(SRPT * NS, TAIL)],
                            out_hbm.at[cid, pl.ds(SRPT * NS, TAIL)])

    return _sc_gather_scatter


def _node_mlp_body(eps_ref, h_ref, a0_ref, a1_ref, w1_ref, b1_ref, w2_ref,
                   b2_ref, out_ref):
    z = (1.0 + eps_ref[0]) * h_ref[...] + a0_ref[...] + a1_ref[...]
    z1 = lax.dot_general(z, w1_ref[...], _CT,
                         preferred_element_type=jnp.float32)
    z1 = jnp.maximum(z1 + b1_ref[...], 0.0)
    z2 = lax.dot_general(z1, w2_ref[...], _CT,
                         preferred_element_type=jnp.float32)
    out_ref[...] = jnp.maximum(z2 + b2_ref[...], 0.0)


def _node_mlp(h, a0, a1, eps_l, w1, b1, w2, b2):
    BN = 2000
    return pl.pallas_call(
        _node_mlp_body,
        grid=(N // BN,),
        in_specs=[
            pl.BlockSpec(memory_space=pltpu.SMEM),
            pl.BlockSpec((BN, D), lambda i: (i, 0)),
            pl.BlockSpec((BN, D), lambda i: (i, 0)),
            pl.BlockSpec((BN, D), lambda i: (i, 0)),
            pl.BlockSpec((D, D), lambda i: (0, 0)),
            pl.BlockSpec((1, D), lambda i: (0, 0)),
            pl.BlockSpec((D, D), lambda i: (0, 0)),
            pl.BlockSpec((1, D), lambda i: (0, 0)),
        ],
        out_specs=pl.BlockSpec((BN, D), lambda i: (i, 0)),
        out_shape=jax.ShapeDtypeStruct((N, D), jnp.float32),
    )(eps_l.reshape(1), h, a0, a1, w1, b1.reshape(1, D), w2, b2.reshape(1, D))


def _s2s_body(h_ref, b_ref, wih_ref, whh_ref, bi_ref, bh_ref, f1w_ref,
              f1b_ref, f4w_ref, f4b_ref, out_ref):
    h = h_ref[...]
    gid = lax.broadcasted_iota(jnp.int32, (G, N), 0)
    maskb = gid == b_ref[...]
    q_star = jnp.zeros((G, 2 * D), jnp.float32)
    hs = jnp.zeros((G, D), jnp.float32)
    cs = jnp.zeros((G, D), jnp.float32)
    for _ in range(6):
        gates = (lax.dot_general(q_star, wih_ref[...], _CT,
                                 preferred_element_type=jnp.float32)
                 + lax.dot_general(hs, whh_ref[...], _CT,
                                   preferred_element_type=jnp.float32)
                 + bi_ref[...] + bh_ref[...])
        ig = jax.nn.sigmoid(gates[:, :D])
        fg = jax.nn.sigmoid(gates[:, D:2 * D])
        gg = jnp.tanh(gates[:, 2 * D:3 * D])
        og = jax.nn.sigmoid(gates[:, 3 * D:])
        cs = fg * cs + ig * gg
        hs = og * jnp.tanh(cs)
        s = lax.dot_general(hs, h, _CT,
                            preferred_element_type=jnp.float32)
        emax = jnp.max(jnp.where(maskb, s, -jnp.inf), axis=1, keepdims=True)
        a = jnp.exp(jnp.where(maskb, s - emax, -jnp.inf))
        denom = jnp.sum(a, axis=1, keepdims=True)
        a = a / (denom + 1e-16)
        r = lax.dot_general(a, h, (((1,), (0,)), ((), ())),
                            preferred_element_type=jnp.float32)
        q_star = jnp.concatenate([hs, r], axis=1)
    z = lax.dot_general(q_star, f1w_ref[...], _CT,
                        preferred_element_type=jnp.float32)
    z = jnp.maximum(z + f1b_ref[...], 0.0)
    out = lax.dot_general(z, f4w_ref[...], _CT,
                          preferred_element_type=jnp.float32)
    out_ref[...] = out + f4b_ref[...]


def _set2set(h, batch_row, lstm_Wih, lstm_Whh, lstm_bih, lstm_bhh,
             fc1_W, fc1_b, fc4_W, fc4_b):
    return pl.pallas_call(
        _s2s_body,
        out_shape=jax.ShapeDtypeStruct((G, C), jnp.float32),
    )(h, batch_row, lstm_Wih, lstm_Whh, lstm_bih.reshape(1, 4 * D),
      lstm_bhh.reshape(1, 4 * D), fc1_W, fc1_b.reshape(1, D), fc4_W,
      fc4_b.reshape(1, C))


def kernel(x, edge_index, edge_attr, batch, bW1, bb1, bW2, bb2, mW1, mb1,
           mW2, mb2, eps, lstm_Wih, lstm_Whh, lstm_bih, lstm_bhh, fc1_W,
           fc1_b, fc4_W, fc4_b):
    src = edge_index[0].astype(jnp.int32).reshape(NW, PASS, CPP, K)
    dst = edge_index[1].astype(jnp.int32).reshape(NW, PASS, CPP, K)
    zer = jnp.zeros((SRPT, D), jnp.float32)
    batch_row = batch.astype(jnp.int32).reshape(1, N)
    h = x
    eembs = [_edge_mlp(edge_attr, bW1[l], bb1[l], bW2[l], bb2[l])
             for l in range(L)]
    for l in range(L):
        parts = _sc_gather_scatter_fn()(h, eembs[l], src, dst, zer)
        h = _node_mlp(h, parts[0], parts[1], eps[l], mW1[l], mb1[l],
                      mW2[l], mb2[l])
    return _set2set(h, batch_row, lstm_Wih, lstm_Whh, lstm_bih, lstm_bhh,
                    fc1_W, fc1_b, fc4_W, fc4_b)
